# baseline (speedup 1.0000x reference)
import functools

import jax
import jax.numpy as jnp
from jax import lax
from jax.experimental import pallas as pl
from jax.experimental.pallas import tpu as pltpu
from jax.experimental.pallas import tpu_sc as plsc

NUM_EXPERTS = 8
HIDDEN = 1024
INTER = 2048
T = 2048

RB = 512
NB = 15
CAP = NB * RB
FT = 512
NFT = INTER // FT

NC = 2
NS = 16
NW = NC * NS
ROWS_W = CAP // NW
GCH = 40
NCH = ROWS_W // GCH
TCH = T // NW
ECH = 16

def _mesh():
    return plsc.VectorSubcoreMesh(core_axis_name="c", subcore_axis_name="s",
                                  num_cores=NC, num_subcores=NS)


def _route_body(logits_ref, pos0_ref, pos1_ref, w0_ref, w1_ref,
                be_ref, ba_ref, xb_ref):
    E = NUM_EXPERTS
    logits = logits_ref[...].astype(jnp.float32)
    iota_e = lax.broadcasted_iota(jnp.int32, (T, E), 1)
    m0 = jnp.max(logits, axis=1, keepdims=True)
    idx0 = jnp.min(jnp.where(logits == m0, iota_e, E), axis=1, keepdims=True)
    masked = jnp.where(iota_e == idx0, -jnp.inf, logits)
    m1 = jnp.max(masked, axis=1, keepdims=True)
    idx1 = jnp.min(jnp.where(masked == m1, iota_e, E), axis=1, keepdims=True)
    w0 = 1.0 / (1.0 + jnp.exp(m1 - m0))
    oh0 = (iota_e == idx0).astype(jnp.float32)
    oh1 = (iota_e == idx1).astype(jnp.float32)

    def excl_cumsum(a):
        s = a
        sh = 1
        while sh < T:
            s = s + jnp.concatenate(
                [jnp.zeros((sh, E), jnp.float32), s[:T - sh]], axis=0)
            sh *= 2
        return s - a

    c0 = excl_cumsum(oh0)
    c1 = excl_cumsum(oh1)
    count0 = jnp.sum(oh0, axis=0, keepdims=True)
    count = count0 + jnp.sum(oh1, axis=0, keepdims=True)
    pc = jnp.ceil(count / RB) * RB
    tri = (lax.broadcasted_iota(jnp.int32, (E, E), 0)
           < lax.broadcasted_iota(jnp.int32, (E, E), 1)).astype(jnp.float32)
    offs = lax.dot_general(pc, tri, (((1,), (0,)), ((), ())),
                           preferred_element_type=jnp.float32)
    total_used = jnp.sum(pc)
    rank0 = jnp.sum(oh0 * c0, axis=1, keepdims=True)
    rank1 = (jnp.sum(oh1 * c1, axis=1, keepdims=True)
             + jnp.sum(oh1 * count0, axis=1, keepdims=True))
    pos0_ref[...] = (jnp.sum(oh0 * offs, axis=1, keepdims=True)
                     + rank0).astype(jnp.int32)
    pos1_ref[...] = (jnp.sum(oh1 * offs, axis=1, keepdims=True)
                     + rank1).astype(jnp.int32)
    w0_ref[...] = w0
    w1_ref[...] = 1.0 - w0

    iota_b = lax.broadcasted_iota(jnp.int32, (1, NB), 1)
    bb = (iota_b * RB).astype(jnp.float32)
    bbase = jnp.minimum(bb, total_used - 1.0)
    acc = jnp.zeros((1, NB), jnp.float32)
    for e in range(E):
        off_e = lax.slice(offs, (0, e), (1, e + 1))
        acc = acc + (bbase >= off_e).astype(jnp.float32)
    be_ref[...] = (acc - 1.0).astype(jnp.int32)
    ba_ref[...] = (bb < total_used).astype(jnp.int32)
    nbt = (total_used / RB).astype(jnp.int32)
    xb_ref[...] = jnp.minimum(iota_b, nbt - 1)


def _route(router_logits):
    return pl.pallas_call(
        _route_body,
        grid=(1,),
        in_specs=[pl.BlockSpec((T, NUM_EXPERTS), lambda i: (0, 0))],
        out_specs=[
            pl.BlockSpec((T, 1), lambda i: (0, 0)),
            pl.BlockSpec((T, 1), lambda i: (0, 0)),
            pl.BlockSpec((T, 1), lambda i: (0, 0)),
            pl.BlockSpec((T, 1), lambda i: (0, 0)),
            pl.BlockSpec((1, NB), lambda i: (0, 0)),
            pl.BlockSpec((1, NB), lambda i: (0, 0)),
            pl.BlockSpec((1, NB), lambda i: (0, 0)),
        ],
        out_shape=[
            jax.ShapeDtypeStruct((T, 1), jnp.int32),
            jax.ShapeDtypeStruct((T, 1), jnp.int32),
            jax.ShapeDtypeStruct((T, 1), jnp.float32),
            jax.ShapeDtypeStruct((T, 1), jnp.float32),
            jax.ShapeDtypeStruct((1, NB), jnp.int32),
            jax.ShapeDtypeStruct((1, NB), jnp.int32),
            jax.ShapeDtypeStruct((1, NB), jnp.int32),
        ],
    )(router_logits)


def _dispatch_body(pos0_hbm, pos1_hbm, w0_hbm, w1_hbm, x_hbm, z32_hbm,
                   zf32_hbm,
                   xs_hbm, rs_hbm,
                   pos0_v, pos1_v, w0_v, w1_v, rt_v, rs_v, idxs_v,
                   rows_a, rows_b, rt_sh,
                   gsem_a, gsem_b, wsem_a, wsem_b):
    c = lax.axis_index("c")
    s = lax.axis_index("s")
    lane = lax.iota(jnp.int32, 16)

    @pl.when(s == 0)
    def _():
        pltpu.sync_copy(pos0_hbm, pos0_v)
        pltpu.sync_copy(pos1_hbm, pos1_v)
        pltpu.sync_copy(z32_hbm, rt_v)

        def scat_body(j, carry):
            tok = lane + j * 16
            plsc.store_scatter(rt_v, [pos0_v[pl.ds(j * 16, 16)]], tok)
            plsc.store_scatter(rt_v, [pos1_v[pl.ds(j * 16, 16)]], tok)
            return carry

        lax.fori_loop(0, T // 16, scat_body, 0)
        pltpu.sync_copy(rt_v, rt_sh)

    @pl.when((s == 1) & (c == 0))
    def _():
        pltpu.sync_copy(pos0_hbm, pos0_v)
        pltpu.sync_copy(pos1_hbm, pos1_v)
        pltpu.sync_copy(w0_hbm, w0_v)
        pltpu.sync_copy(w1_hbm, w1_v)
        pltpu.sync_copy(zf32_hbm, rs_v)

        def scat_body(j, carry):
            plsc.store_scatter(rs_v, [pos0_v[pl.ds(j * 16, 16)]],
                               w0_v[pl.ds(j * 16, 16)])
            plsc.store_scatter(rs_v, [pos1_v[pl.ds(j * 16, 16)]],
                               w1_v[pl.ds(j * 16, 16)])
            return carry

        lax.fori_loop(0, T // 16, scat_body, 0)
        pltpu.sync_copy(rs_v, rs_hbm)

    plsc.subcore_barrier()
    w = s * NC + c
    base = w * ROWS_W
    pltpu.sync_copy(rt_sh.at[pl.ds(base, ROWS_W)], idxs_v)

    bufs = (rows_a, rows_b)
    gsems = (gsem_a, gsem_b)
    wsems = (wsem_a, wsem_b)
    gths = [None] * NCH
    wrs = [None] * NCH
    for i in range(NCH):
        b = i % 2
        if i >= 2:
            wrs[i - 2].wait()
        gths[i] = pltpu.async_copy(
            x_hbm.at[idxs_v.at[pl.ds(i * GCH, GCH)]], bufs[b], gsems[b])
        if i >= 1:
            gths[i - 1].wait()
            wrs[i - 1] = pltpu.async_copy(
                bufs[(i - 1) % 2],
                xs_hbm.at[pl.ds(base + (i - 1) * GCH, GCH)],
                wsems[(i - 1) % 2])
    gths[NCH - 1].wait()
    wrs[NCH - 1] = pltpu.async_copy(
        bufs[(NCH - 1) % 2], xs_hbm.at[pl.ds(base + (NCH - 1) * GCH, GCH)],
        wsems[(NCH - 1) % 2])
    wrs[NCH - 2].wait()
    wrs[NCH - 1].wait()


_dispatch_impl = None


def _dispatch(pos0f, pos1f, w0f, w1f, x3d, z32, zf32):
    global _dispatch_impl
    if _dispatch_impl is None:
        _dispatch_impl = pl.kernel(
            _dispatch_body,
            out_type=[
                jax.ShapeDtypeStruct((CAP, HIDDEN // 2), jnp.int32),
                jax.ShapeDtypeStruct((CAP,), jnp.float32),
            ],
            mesh=_mesh(),
            scratch_types=[
                pltpu.VMEM((T,), jnp.int32),
                pltpu.VMEM((T,), jnp.int32),
                pltpu.VMEM((T,), jnp.float32),
                pltpu.VMEM((T,), jnp.float32),
                pltpu.VMEM((CAP,), jnp.int32),
                pltpu.VMEM((CAP,), jnp.float32),
                pltpu.VMEM((ROWS_W,), jnp.int32),
                pltpu.VMEM((GCH, HIDDEN // 2), jnp.int32),
                pltpu.VMEM((GCH, HIDDEN // 2), jnp.int32),
                pltpu.VMEM_SHARED((CAP,), jnp.int32),
                pltpu.SemaphoreType.DMA,
                pltpu.SemaphoreType.DMA,
                pltpu.SemaphoreType.DMA,
                pltpu.SemaphoreType.DMA,
            ],
            compiler_params=pltpu.CompilerParams(needs_layout_passes=False),
        )
    return _dispatch_impl(pos0f, pos1f, w0f, w1f, x3d, z32, zf32)


def _gmm_body(be_ref, ba_ref, xb_ref,
              xs_ref, rs_ref, w13g_ref, w13u_ref, w2_ref, y_ref):
    b = pl.program_id(0)
    f = pl.program_id(1)

    @pl.when(ba_ref[b] == 1)
    def _():
        x = xs_ref[...].astype(jnp.bfloat16)
        gate = lax.dot_general(x, w13g_ref[0], (((1,), (1,)), ((), ())),
                               preferred_element_type=jnp.float32)
        up = lax.dot_general(x, w13u_ref[0], (((1,), (1,)), ((), ())),
                             preferred_element_type=jnp.float32)
        act = (gate * jax.nn.sigmoid(gate) * up).astype(jnp.bfloat16)
        part = lax.dot_general(act, w2_ref[0], (((1,), (1,)), ((), ())),
                               preferred_element_type=jnp.float32)

        @pl.when(f == 0)
        def _():
            y_ref[...] = part

        @pl.when(f > 0)
        def _():
            y_ref[...] += part

        @pl.when(f == NFT - 1)
        def _():
            y_ref[...] *= rs_ref[...]


def _feff(ba_ref, b, f):
    return jnp.where(ba_ref[b] == 0, NFT - 1, f)


def _gmm(bev, bav, xbv, xs, rs, w13_16, w2_16):
    grid_spec = pltpu.PrefetchScalarGridSpec(
        num_scalar_prefetch=3,
        grid=(NB, NFT),
        in_specs=[
            pl.BlockSpec((RB, HIDDEN), lambda b, f, be, ba, xb: (xb[b], 0)),
            pl.BlockSpec((RB, 1), lambda b, f, be, ba, xb: (xb[b], 0)),
            pl.BlockSpec((1, FT, HIDDEN),
                         lambda b, f, be, ba, xb: (be[b], _feff(ba, b, f), 0)),
            pl.BlockSpec((1, FT, HIDDEN),
                         lambda b, f, be, ba, xb:
                         (be[b], NFT + _feff(ba, b, f), 0)),
            pl.BlockSpec((1, HIDDEN, FT),
                         lambda b, f, be, ba, xb: (be[b], 0, _feff(ba, b, f))),
        ],
        out_specs=pl.BlockSpec((RB, HIDDEN), lambda b, f, be, ba, xb: (b, 0)),
    )
    return pl.pallas_call(
        _gmm_body,
        grid_spec=grid_spec,
        out_shape=jax.ShapeDtypeStruct((CAP, HIDDEN), jnp.float32),
        compiler_params=pltpu.CompilerParams(
            dimension_semantics=("arbitrary", "arbitrary"),
        ),
    )(bev, bav, xbv, xs, rs, w13_16, w13_16, w2_16)


def _combine_body(pos0_hbm, pos1_hbm, y_hbm, out_hbm,
             i0_v, i1_v, r0_v, r1_v, s0, s1):
    c = lax.axis_index("c")
    s = lax.axis_index("s")
    w = s * NC + c
    tb = w * TCH
    pltpu.sync_copy(pos0_hbm.at[pl.ds(tb, TCH)], i0_v)
    pltpu.sync_copy(pos1_hbm.at[pl.ds(tb, TCH)], i1_v)

    def chunk_body(ci, carry):
        cp0 = pltpu.async_copy(
            y_hbm.at[i0_v.at[pl.ds(ci * ECH, ECH)]], r0_v, s0)
        cp1 = pltpu.async_copy(
            y_hbm.at[i1_v.at[pl.ds(ci * ECH, ECH)]], r1_v, s1)
        cp0.wait()
        cp1.wait()

        def row_body(i, carry2):
            def col_body(j, carry3):
                r0_v[i, pl.ds(j * 16, 16)] = (
                    r0_v[i, pl.ds(j * 16, 16)] + r1_v[i, pl.ds(j * 16, 16)])
                return carry3
            lax.fori_loop(0, HIDDEN // 16, col_body, 0)
            return carry2

        lax.fori_loop(0, ECH, row_body, 0)
        pltpu.sync_copy(r0_v, out_hbm.at[pl.ds(tb + ci * ECH, ECH)])
        return carry

    lax.fori_loop(0, TCH // ECH, chunk_body, 0)


_combine_impl = None


def _combine(pos0f, pos1f, y):
    global _combine_impl
    if _combine_impl is None:
        _combine_impl = pl.kernel(
            _combine_body,
            out_type=jax.ShapeDtypeStruct((T, HIDDEN), jnp.float32),
            mesh=_mesh(),
            scratch_types=[
                pltpu.VMEM((TCH,), jnp.int32),
                pltpu.VMEM((TCH,), jnp.int32),
                pltpu.VMEM((ECH, HIDDEN), jnp.float32),
                pltpu.VMEM((ECH, HIDDEN), jnp.float32),
                pltpu.SemaphoreType.DMA,
                pltpu.SemaphoreType.DMA,
            ],
            compiler_params=pltpu.CompilerParams(needs_layout_passes=False),
        )
    return _combine_impl(pos0f, pos1f, y)


def kernel(hidden_states, router_logits, w13_weight, w2_weight):
    w13_16 = w13_weight.astype(jnp.bfloat16)
    w2_16 = w2_weight.astype(jnp.bfloat16)

    pos0, pos1, w0, w1, be, ba, xb = _route(router_logits)
    pos0f = pos0.reshape(T)
    pos1f = pos1.reshape(T)

    x16 = hidden_states.astype(jnp.bfloat16)
    x_i32 = lax.bitcast_convert_type(
        x16.reshape(T, HIDDEN // 2, 2), jnp.int32)
    z32 = jnp.zeros((CAP,), jnp.int32)
    zf32 = jnp.zeros((CAP,), jnp.float32)
    xs, rs = _dispatch(pos0f, pos1f, w0.reshape(T), w1.reshape(T),
                       x_i32, z32, zf32)
    xs16 = lax.bitcast_convert_type(xs, jnp.bfloat16).reshape(CAP, HIDDEN)
    y = _gmm(be.reshape(NB), ba.reshape(NB), xb.reshape(NB),
             xs16, rs.reshape(CAP, 1), w13_16, w2_16)
    return _combine(pos0f, pos1f, y)


# speedup vs baseline: 1.3574x; 1.3574x over previous
import functools

import jax
import jax.numpy as jnp
from jax import lax
from jax.experimental import pallas as pl
from jax.experimental.pallas import tpu as pltpu
from jax.experimental.pallas import tpu_sc as plsc

NUM_EXPERTS = 8
HIDDEN = 1024
INTER = 2048
T = 2048

RB = 512
NB = 15
CAP = NB * RB
FT = 512
NFT = INTER // FT

NC = 2
NS = 16
NW = NC * NS
ROWS_W = CAP // NW
GCH = 40
NCH = ROWS_W // GCH
TCH = T // NW
ECH = 16

def _mesh():
    return plsc.VectorSubcoreMesh(core_axis_name="c", subcore_axis_name="s",
                                  num_cores=NC, num_subcores=NS)


def _route_body(logits_ref, pos0_ref, pos1_ref, w0_ref, w1_ref,
                be_ref, ba_ref, xb_ref):
    E = NUM_EXPERTS
    logits = logits_ref[...].astype(jnp.float32)
    iota_e = lax.broadcasted_iota(jnp.int32, (T, E), 1)
    m0 = jnp.max(logits, axis=1, keepdims=True)
    idx0 = jnp.min(jnp.where(logits == m0, iota_e, E), axis=1, keepdims=True)
    masked = jnp.where(iota_e == idx0, -jnp.inf, logits)
    m1 = jnp.max(masked, axis=1, keepdims=True)
    idx1 = jnp.min(jnp.where(masked == m1, iota_e, E), axis=1, keepdims=True)
    w0 = 1.0 / (1.0 + jnp.exp(m1 - m0))
    oh0 = (iota_e == idx0).astype(jnp.float32)
    oh1 = (iota_e == idx1).astype(jnp.float32)

    def excl_cumsum(a):
        s = a
        sh = 1
        while sh < T:
            s = s + jnp.concatenate(
                [jnp.zeros((sh, E), jnp.float32), s[:T - sh]], axis=0)
            sh *= 2
        return s - a

    c0 = excl_cumsum(oh0)
    c1 = excl_cumsum(oh1)
    count0 = jnp.sum(oh0, axis=0, keepdims=True)
    count = count0 + jnp.sum(oh1, axis=0, keepdims=True)
    pc = jnp.ceil(count / RB) * RB
    tri = (lax.broadcasted_iota(jnp.int32, (E, E), 0)
           < lax.broadcasted_iota(jnp.int32, (E, E), 1)).astype(jnp.float32)
    offs = lax.dot_general(pc, tri, (((1,), (0,)), ((), ())),
                           preferred_element_type=jnp.float32)
    total_used = jnp.sum(pc)
    rank0 = jnp.sum(oh0 * c0, axis=1, keepdims=True)
    rank1 = (jnp.sum(oh1 * c1, axis=1, keepdims=True)
             + jnp.sum(oh1 * count0, axis=1, keepdims=True))
    pos0_ref[...] = (jnp.sum(oh0 * offs, axis=1, keepdims=True)
                     + rank0).astype(jnp.int32)
    pos1_ref[...] = (jnp.sum(oh1 * offs, axis=1, keepdims=True)
                     + rank1).astype(jnp.int32)
    w0_ref[...] = w0
    w1_ref[...] = 1.0 - w0

    iota_b = lax.broadcasted_iota(jnp.int32, (1, NB), 1)
    bb = (iota_b * RB).astype(jnp.float32)
    bbase = jnp.minimum(bb, total_used - 1.0)
    acc = jnp.zeros((1, NB), jnp.float32)
    for e in range(E):
        off_e = lax.slice(offs, (0, e), (1, e + 1))
        acc = acc + (bbase >= off_e).astype(jnp.float32)
    be_ref[...] = (acc - 1.0).astype(jnp.int32)
    ba_ref[...] = (bb < total_used).astype(jnp.int32)
    nbt = (total_used / RB).astype(jnp.int32)
    xb_ref[...] = jnp.minimum(iota_b, nbt - 1)


def _route(router_logits):
    return pl.pallas_call(
        _route_body,
        grid=(1,),
        in_specs=[pl.BlockSpec((T, NUM_EXPERTS), lambda i: (0, 0))],
        out_specs=[
            pl.BlockSpec((T, 1), lambda i: (0, 0)),
            pl.BlockSpec((T, 1), lambda i: (0, 0)),
            pl.BlockSpec((T, 1), lambda i: (0, 0)),
            pl.BlockSpec((T, 1), lambda i: (0, 0)),
            pl.BlockSpec((1, NB), lambda i: (0, 0)),
            pl.BlockSpec((1, NB), lambda i: (0, 0)),
            pl.BlockSpec((1, NB), lambda i: (0, 0)),
        ],
        out_shape=[
            jax.ShapeDtypeStruct((T, 1), jnp.int32),
            jax.ShapeDtypeStruct((T, 1), jnp.int32),
            jax.ShapeDtypeStruct((T, 1), jnp.float32),
            jax.ShapeDtypeStruct((T, 1), jnp.float32),
            jax.ShapeDtypeStruct((1, NB), jnp.int32),
            jax.ShapeDtypeStruct((1, NB), jnp.int32),
            jax.ShapeDtypeStruct((1, NB), jnp.int32),
        ],
    )(router_logits)


def _dispatch_body(pos0_hbm, pos1_hbm, w0_hbm, w1_hbm, x_hbm, z32_hbm,
                   zf32_hbm,
                   xs_hbm, rs_hbm,
                   pos0_v, pos1_v, w0_v, w1_v, rt_v, rs_v, idxs_v,
                   rows_a, rows_b, rt_sh,
                   gsem_a, gsem_b, wsem_a, wsem_b):
    c = lax.axis_index("c")
    s = lax.axis_index("s")
    lane = lax.iota(jnp.int32, 16)

    @pl.when(s == 0)
    def _():
        pltpu.sync_copy(pos0_hbm, pos0_v)
        pltpu.sync_copy(pos1_hbm, pos1_v)
        pltpu.sync_copy(z32_hbm, rt_v)

        def scat_body(j, carry):
            tok = lane + j * 16
            plsc.store_scatter(rt_v, [pos0_v[pl.ds(j * 16, 16)]], tok)
            plsc.store_scatter(rt_v, [pos1_v[pl.ds(j * 16, 16)]], tok)
            return carry

        lax.fori_loop(0, T // 16, scat_body, 0)
        pltpu.sync_copy(rt_v, rt_sh)

    @pl.when((s == 1) & (c == 0))
    def _():
        pltpu.sync_copy(pos0_hbm, pos0_v)
        pltpu.sync_copy(pos1_hbm, pos1_v)
        pltpu.sync_copy(w0_hbm, w0_v)
        pltpu.sync_copy(w1_hbm, w1_v)
        pltpu.sync_copy(zf32_hbm, rs_v)

        def scat_body(j, carry):
            plsc.store_scatter(rs_v, [pos0_v[pl.ds(j * 16, 16)]],
                               w0_v[pl.ds(j * 16, 16)])
            plsc.store_scatter(rs_v, [pos1_v[pl.ds(j * 16, 16)]],
                               w1_v[pl.ds(j * 16, 16)])
            return carry

        lax.fori_loop(0, T // 16, scat_body, 0)
        pltpu.sync_copy(rs_v, rs_hbm)

    plsc.subcore_barrier()
    w = s * NC + c
    base = w * ROWS_W
    pltpu.sync_copy(rt_sh.at[pl.ds(base, ROWS_W)], idxs_v)

    bufs = (rows_a, rows_b)
    gsems = (gsem_a, gsem_b)
    wsems = (wsem_a, wsem_b)
    gths = [None] * NCH
    wrs = [None] * NCH
    for i in range(NCH):
        b = i % 2
        if i >= 2:
            wrs[i - 2].wait()
        gths[i] = pltpu.async_copy(
            x_hbm.at[idxs_v.at[pl.ds(i * GCH, GCH)]], bufs[b], gsems[b])
        if i >= 1:
            gths[i - 1].wait()
            wrs[i - 1] = pltpu.async_copy(
                bufs[(i - 1) % 2],
                xs_hbm.at[pl.ds(base + (i - 1) * GCH, GCH)],
                wsems[(i - 1) % 2])
    gths[NCH - 1].wait()
    wrs[NCH - 1] = pltpu.async_copy(
        bufs[(NCH - 1) % 2], xs_hbm.at[pl.ds(base + (NCH - 1) * GCH, GCH)],
        wsems[(NCH - 1) % 2])
    wrs[NCH - 2].wait()
    wrs[NCH - 1].wait()


_dispatch_impl = None


def _dispatch(pos0f, pos1f, w0f, w1f, x3d, z32, zf32):
    global _dispatch_impl
    if _dispatch_impl is None:
        _dispatch_impl = pl.kernel(
            _dispatch_body,
            out_type=[
                jax.ShapeDtypeStruct((CAP, HIDDEN), jnp.float32),
                jax.ShapeDtypeStruct((CAP,), jnp.float32),
            ],
            mesh=_mesh(),
            scratch_types=[
                pltpu.VMEM((T,), jnp.int32),
                pltpu.VMEM((T,), jnp.int32),
                pltpu.VMEM((T,), jnp.float32),
                pltpu.VMEM((T,), jnp.float32),
                pltpu.VMEM((CAP,), jnp.int32),
                pltpu.VMEM((CAP,), jnp.float32),
                pltpu.VMEM((ROWS_W,), jnp.int32),
                pltpu.VMEM((GCH, HIDDEN), jnp.float32),
                pltpu.VMEM((GCH, HIDDEN), jnp.float32),
                pltpu.VMEM_SHARED((CAP,), jnp.int32),
                pltpu.SemaphoreType.DMA,
                pltpu.SemaphoreType.DMA,
                pltpu.SemaphoreType.DMA,
                pltpu.SemaphoreType.DMA,
            ],
            compiler_params=pltpu.CompilerParams(needs_layout_passes=False),
        )
    return _dispatch_impl(pos0f, pos1f, w0f, w1f, x3d, z32, zf32)


def _gmm_body(be_ref, ba_ref, xb_ref,
              xs_ref, rs_ref, w13g_ref, w13u_ref, w2_ref, y_ref):
    b = pl.program_id(0)
    f = pl.program_id(1)

    @pl.when(ba_ref[b] == 1)
    def _():
        x = xs_ref[...].astype(jnp.bfloat16)
        gate = lax.dot_general(x, w13g_ref[0], (((1,), (1,)), ((), ())),
                               preferred_element_type=jnp.float32)
        up = lax.dot_general(x, w13u_ref[0], (((1,), (1,)), ((), ())),
                             preferred_element_type=jnp.float32)
        act = (gate * jax.nn.sigmoid(gate) * up).astype(jnp.bfloat16)
        part = lax.dot_general(act, w2_ref[0], (((1,), (1,)), ((), ())),
                               preferred_element_type=jnp.float32)

        @pl.when(f == 0)
        def _():
            y_ref[...] = part

        @pl.when(f > 0)
        def _():
            y_ref[...] += part

        @pl.when(f == NFT - 1)
        def _():
            y_ref[...] *= rs_ref[...]


def _feff(ba_ref, b, f):
    return jnp.where(ba_ref[b] == 0, NFT - 1, f)


def _gmm(bev, bav, xbv, xs, rs, w13_16, w2_16):
    grid_spec = pltpu.PrefetchScalarGridSpec(
        num_scalar_prefetch=3,
        grid=(NB, NFT),
        in_specs=[
            pl.BlockSpec((RB, HIDDEN), lambda b, f, be, ba, xb: (xb[b], 0)),
            pl.BlockSpec((RB, 1), lambda b, f, be, ba, xb: (xb[b], 0)),
            pl.BlockSpec((1, FT, HIDDEN),
                         lambda b, f, be, ba, xb: (be[b], _feff(ba, b, f), 0)),
            pl.BlockSpec((1, FT, HIDDEN),
                         lambda b, f, be, ba, xb:
                         (be[b], NFT + _feff(ba, b, f), 0)),
            pl.BlockSpec((1, HIDDEN, FT),
                         lambda b, f, be, ba, xb: (be[b], 0, _feff(ba, b, f))),
        ],
        out_specs=pl.BlockSpec((RB, HIDDEN), lambda b, f, be, ba, xb: (b, 0)),
    )
    return pl.pallas_call(
        _gmm_body,
        grid_spec=grid_spec,
        out_shape=jax.ShapeDtypeStruct((CAP, HIDDEN), jnp.float32),
        compiler_params=pltpu.CompilerParams(
            dimension_semantics=("arbitrary", "arbitrary"),
        ),
    )(bev, bav, xbv, xs, rs, w13_16, w13_16, w2_16)


def _combine_body(pos0_hbm, pos1_hbm, y_hbm, out_hbm,
             i0_v, i1_v, r0_v, r1_v, s0, s1):
    c = lax.axis_index("c")
    s = lax.axis_index("s")
    w = s * NC + c
    tb = w * TCH
    pltpu.sync_copy(pos0_hbm.at[pl.ds(tb, TCH)], i0_v)
    pltpu.sync_copy(pos1_hbm.at[pl.ds(tb, TCH)], i1_v)

    def chunk_body(ci, carry):
        cp0 = pltpu.async_copy(
            y_hbm.at[i0_v.at[pl.ds(ci * ECH, ECH)]], r0_v, s0)
        cp1 = pltpu.async_copy(
            y_hbm.at[i1_v.at[pl.ds(ci * ECH, ECH)]], r1_v, s1)
        cp0.wait()
        cp1.wait()

        def row_body(i, carry2):
            def col_body(j, carry3):
                r0_v[i, pl.ds(j * 16, 16)] = (
                    r0_v[i, pl.ds(j * 16, 16)] + r1_v[i, pl.ds(j * 16, 16)])
                return carry3
            lax.fori_loop(0, HIDDEN // 16, col_body, 0)
            return carry2

        lax.fori_loop(0, ECH, row_body, 0)
        pltpu.sync_copy(r0_v, out_hbm.at[pl.ds(tb + ci * ECH, ECH)])
        return carry

    lax.fori_loop(0, TCH // ECH, chunk_body, 0)


_combine_impl = None


def _combine(pos0f, pos1f, y):
    global _combine_impl
    if _combine_impl is None:
        _combine_impl = pl.kernel(
            _combine_body,
            out_type=jax.ShapeDtypeStruct((T, HIDDEN), jnp.float32),
            mesh=_mesh(),
            scratch_types=[
                pltpu.VMEM((TCH,), jnp.int32),
                pltpu.VMEM((TCH,), jnp.int32),
                pltpu.VMEM((ECH, HIDDEN), jnp.float32),
                pltpu.VMEM((ECH, HIDDEN), jnp.float32),
                pltpu.SemaphoreType.DMA,
                pltpu.SemaphoreType.DMA,
            ],
            compiler_params=pltpu.CompilerParams(needs_layout_passes=False),
        )
    return _combine_impl(pos0f, pos1f, y)


def kernel(hidden_states, router_logits, w13_weight, w2_weight):
    w13_16 = w13_weight.astype(jnp.bfloat16)
    w2_16 = w2_weight.astype(jnp.bfloat16)

    pos0, pos1, w0, w1, be, ba, xb = _route(router_logits)
    pos0f = pos0.reshape(T)
    pos1f = pos1.reshape(T)

    z32 = jnp.zeros((CAP,), jnp.int32)
    zf32 = jnp.zeros((CAP,), jnp.float32)
    xs, rs = _dispatch(pos0f, pos1f, w0.reshape(T), w1.reshape(T),
                       hidden_states, z32, zf32)
    y = _gmm(be.reshape(NB), ba.reshape(NB), xb.reshape(NB),
             xs, rs.reshape(CAP, 1), w13_16, w2_16)
    return _combine(pos0f, pos1f, y)


# speedup vs baseline: 1.4100x; 1.0387x over previous
import functools

import jax
import jax.numpy as jnp
from jax import lax
from jax.experimental import pallas as pl
from jax.experimental.pallas import tpu as pltpu
from jax.experimental.pallas import tpu_sc as plsc

NUM_EXPERTS = 8
HIDDEN = 1024
INTER = 2048
T = 2048

RB = 512
NB = 15
CAP = NB * RB
FT = 512
NFT = INTER // FT

NC = 2
NS = 16
NW = NC * NS
ROWS_W = CAP // NW
GCH = 40
NCH = ROWS_W // GCH
TCH = T // NW
ECH = 16

def _mesh():
    return plsc.VectorSubcoreMesh(core_axis_name="c", subcore_axis_name="s",
                                  num_cores=NC, num_subcores=NS)


def _route_body(logits_ref, x_ref, pos0_ref, pos1_ref, w0_ref, w1_ref,
                be_ref, ba_ref, xb_ref, xp_ref):
    E = NUM_EXPERTS
    logits = logits_ref[...].astype(jnp.float32)

    xb = lax.bitcast_convert_type(x_ref[...], jnp.int32)

    def rne(v):
        return v + jnp.int32(0x7FFF) + ((v >> 16) & jnp.int32(1))

    r_e = rne(xb[:, :HIDDEN // 2])
    r_o = rne(xb[:, HIDDEN // 2:])
    xp_ref[...] = ((r_o & jnp.int32(-65536))
                   | ((r_e >> 16) & jnp.int32(0xFFFF)))
    iota_e = lax.broadcasted_iota(jnp.int32, (T, E), 1)
    m0 = jnp.max(logits, axis=1, keepdims=True)
    idx0 = jnp.min(jnp.where(logits == m0, iota_e, E), axis=1, keepdims=True)
    masked = jnp.where(iota_e == idx0, -jnp.inf, logits)
    m1 = jnp.max(masked, axis=1, keepdims=True)
    idx1 = jnp.min(jnp.where(masked == m1, iota_e, E), axis=1, keepdims=True)
    w0 = 1.0 / (1.0 + jnp.exp(m1 - m0))
    oh0 = (iota_e == idx0).astype(jnp.float32)
    oh1 = (iota_e == idx1).astype(jnp.float32)

    def excl_cumsum(a):
        s = a
        sh = 1
        while sh < T:
            s = s + jnp.concatenate(
                [jnp.zeros((sh, E), jnp.float32), s[:T - sh]], axis=0)
            sh *= 2
        return s - a

    c0 = excl_cumsum(oh0)
    c1 = excl_cumsum(oh1)
    count0 = jnp.sum(oh0, axis=0, keepdims=True)
    count = count0 + jnp.sum(oh1, axis=0, keepdims=True)
    pc = jnp.ceil(count / RB) * RB
    tri = (lax.broadcasted_iota(jnp.int32, (E, E), 0)
           < lax.broadcasted_iota(jnp.int32, (E, E), 1)).astype(jnp.float32)
    offs = lax.dot_general(pc, tri, (((1,), (0,)), ((), ())),
                           preferred_element_type=jnp.float32)
    total_used = jnp.sum(pc)
    rank0 = jnp.sum(oh0 * c0, axis=1, keepdims=True)
    rank1 = (jnp.sum(oh1 * c1, axis=1, keepdims=True)
             + jnp.sum(oh1 * count0, axis=1, keepdims=True))
    pos0_ref[...] = (jnp.sum(oh0 * offs, axis=1, keepdims=True)
                     + rank0).astype(jnp.int32)
    pos1_ref[...] = (jnp.sum(oh1 * offs, axis=1, keepdims=True)
                     + rank1).astype(jnp.int32)
    w0_ref[...] = w0
    w1_ref[...] = 1.0 - w0

    iota_b = lax.broadcasted_iota(jnp.int32, (1, NB), 1)
    bb = (iota_b * RB).astype(jnp.float32)
    bbase = jnp.minimum(bb, total_used - 1.0)
    acc = jnp.zeros((1, NB), jnp.float32)
    for e in range(E):
        off_e = lax.slice(offs, (0, e), (1, e + 1))
        acc = acc + (bbase >= off_e).astype(jnp.float32)
    be_ref[...] = (acc - 1.0).astype(jnp.int32)
    ba_ref[...] = (bb < total_used).astype(jnp.int32)
    nbt = (total_used / RB).astype(jnp.int32)
    xb_ref[...] = jnp.minimum(iota_b, nbt - 1)


def _route(router_logits, hidden_states):
    return pl.pallas_call(
        _route_body,
        grid=(1,),
        in_specs=[
            pl.BlockSpec((T, NUM_EXPERTS), lambda i: (0, 0)),
            pl.BlockSpec((T, HIDDEN), lambda i: (0, 0)),
        ],
        out_specs=[
            pl.BlockSpec((T, 1), lambda i: (0, 0)),
            pl.BlockSpec((T, 1), lambda i: (0, 0)),
            pl.BlockSpec((T, 1), lambda i: (0, 0)),
            pl.BlockSpec((T, 1), lambda i: (0, 0)),
            pl.BlockSpec((1, NB), lambda i: (0, 0)),
            pl.BlockSpec((1, NB), lambda i: (0, 0)),
            pl.BlockSpec((1, NB), lambda i: (0, 0)),
            pl.BlockSpec((T, HIDDEN // 2), lambda i: (0, 0)),
        ],
        out_shape=[
            jax.ShapeDtypeStruct((T, 1), jnp.int32),
            jax.ShapeDtypeStruct((T, 1), jnp.int32),
            jax.ShapeDtypeStruct((T, 1), jnp.float32),
            jax.ShapeDtypeStruct((T, 1), jnp.float32),
            jax.ShapeDtypeStruct((1, NB), jnp.int32),
            jax.ShapeDtypeStruct((1, NB), jnp.int32),
            jax.ShapeDtypeStruct((1, NB), jnp.int32),
            jax.ShapeDtypeStruct((T, HIDDEN // 2), jnp.int32),
        ],
    )(router_logits, hidden_states)


def _dispatch_body(pos0_hbm, pos1_hbm, w0_hbm, w1_hbm, x_hbm, z32_hbm,
                   zf32_hbm,
                   xs_hbm, rs_hbm,
                   pos0_v, pos1_v, w0_v, w1_v, rt_v, rs_v, idxs_v,
                   rows_a, rows_b, rt_sh,
                   gsem_a, gsem_b, wsem_a, wsem_b):
    c = lax.axis_index("c")
    s = lax.axis_index("s")
    lane = lax.iota(jnp.int32, 16)

    @pl.when(s == 0)
    def _():
        pltpu.sync_copy(pos0_hbm, pos0_v)
        pltpu.sync_copy(pos1_hbm, pos1_v)
        pltpu.sync_copy(z32_hbm, rt_v)

        def scat_body(j, carry):
            tok = lane + j * 16
            plsc.store_scatter(rt_v, [pos0_v[pl.ds(j * 16, 16)]], tok)
            plsc.store_scatter(rt_v, [pos1_v[pl.ds(j * 16, 16)]], tok)
            return carry

        lax.fori_loop(0, T // 16, scat_body, 0)
        pltpu.sync_copy(rt_v, rt_sh)

    @pl.when((s == 1) & (c == 0))
    def _():
        pltpu.sync_copy(pos0_hbm, pos0_v)
        pltpu.sync_copy(pos1_hbm, pos1_v)
        pltpu.sync_copy(w0_hbm, w0_v)
        pltpu.sync_copy(w1_hbm, w1_v)
        pltpu.sync_copy(zf32_hbm, rs_v)

        def scat_body(j, carry):
            plsc.store_scatter(rs_v, [pos0_v[pl.ds(j * 16, 16)]],
                               w0_v[pl.ds(j * 16, 16)])
            plsc.store_scatter(rs_v, [pos1_v[pl.ds(j * 16, 16)]],
                               w1_v[pl.ds(j * 16, 16)])
            return carry

        lax.fori_loop(0, T // 16, scat_body, 0)
        pltpu.sync_copy(rs_v, rs_hbm)

    plsc.subcore_barrier()
    w = s * NC + c
    base = w * ROWS_W
    pltpu.sync_copy(rt_sh.at[pl.ds(base, ROWS_W)], idxs_v)

    bufs = (rows_a, rows_b)
    gsems = (gsem_a, gsem_b)
    wsems = (wsem_a, wsem_b)
    gths = [None] * NCH
    wrs = [None] * NCH
    for i in range(NCH):
        b = i % 2
        if i >= 2:
            wrs[i - 2].wait()
        gths[i] = pltpu.async_copy(
            x_hbm.at[idxs_v.at[pl.ds(i * GCH, GCH)]], bufs[b], gsems[b])
        if i >= 1:
            gths[i - 1].wait()
            wrs[i - 1] = pltpu.async_copy(
                bufs[(i - 1) % 2],
                xs_hbm.at[pl.ds(base + (i - 1) * GCH, GCH)],
                wsems[(i - 1) % 2])
    gths[NCH - 1].wait()
    wrs[NCH - 1] = pltpu.async_copy(
        bufs[(NCH - 1) % 2], xs_hbm.at[pl.ds(base + (NCH - 1) * GCH, GCH)],
        wsems[(NCH - 1) % 2])
    wrs[NCH - 2].wait()
    wrs[NCH - 1].wait()


_dispatch_impl = None


def _dispatch(pos0f, pos1f, w0f, w1f, x3d, z32, zf32):
    global _dispatch_impl
    if _dispatch_impl is None:
        _dispatch_impl = pl.kernel(
            _dispatch_body,
            out_type=[
                jax.ShapeDtypeStruct((CAP, HIDDEN // 2), jnp.int32),
                jax.ShapeDtypeStruct((CAP,), jnp.float32),
            ],
            mesh=_mesh(),
            scratch_types=[
                pltpu.VMEM((T,), jnp.int32),
                pltpu.VMEM((T,), jnp.int32),
                pltpu.VMEM((T,), jnp.float32),
                pltpu.VMEM((T,), jnp.float32),
                pltpu.VMEM((CAP,), jnp.int32),
                pltpu.VMEM((CAP,), jnp.float32),
                pltpu.VMEM((ROWS_W,), jnp.int32),
                pltpu.VMEM((GCH, HIDDEN // 2), jnp.int32),
                pltpu.VMEM((GCH, HIDDEN // 2), jnp.int32),
                pltpu.VMEM_SHARED((CAP,), jnp.int32),
                pltpu.SemaphoreType.DMA,
                pltpu.SemaphoreType.DMA,
                pltpu.SemaphoreType.DMA,
                pltpu.SemaphoreType.DMA,
            ],
            compiler_params=pltpu.CompilerParams(needs_layout_passes=False),
        )
    return _dispatch_impl(pos0f, pos1f, w0f, w1f, x3d, z32, zf32)


def _gmm_body(be_ref, ba_ref, xb_ref,
              xs_ref, rs_ref, w13g_ref, w13u_ref, w2_ref, y_ref):
    b = pl.program_id(0)
    f = pl.program_id(1)

    @pl.when(ba_ref[b] == 1)
    def _():
        xi = xs_ref[...]
        x_lo = lax.bitcast_convert_type(xi << 16, jnp.float32)
        x_hi = lax.bitcast_convert_type(xi & jnp.int32(-65536), jnp.float32)
        x = jnp.concatenate([x_lo, x_hi], axis=1).astype(jnp.bfloat16)
        gate = lax.dot_general(x, w13g_ref[0], (((1,), (1,)), ((), ())),
                               preferred_element_type=jnp.float32)
        up = lax.dot_general(x, w13u_ref[0], (((1,), (1,)), ((), ())),
                             preferred_element_type=jnp.float32)
        act = (gate * jax.nn.sigmoid(gate) * up).astype(jnp.bfloat16)
        part = lax.dot_general(act, w2_ref[0], (((1,), (1,)), ((), ())),
                               preferred_element_type=jnp.float32)

        @pl.when(f == 0)
        def _():
            y_ref[...] = part

        @pl.when(f > 0)
        def _():
            y_ref[...] += part

        @pl.when(f == NFT - 1)
        def _():
            y_ref[...] *= rs_ref[...]


def _feff(ba_ref, b, f):
    return jnp.where(ba_ref[b] == 0, NFT - 1, f)


def _gmm(bev, bav, xbv, xs, rs, w13_16, w2_16):
    grid_spec = pltpu.PrefetchScalarGridSpec(
        num_scalar_prefetch=3,
        grid=(NB, NFT),
        in_specs=[
            pl.BlockSpec((RB, HIDDEN // 2),
                         lambda b, f, be, ba, xb: (xb[b], 0)),
            pl.BlockSpec((RB, 1), lambda b, f, be, ba, xb: (xb[b], 0)),
            pl.BlockSpec((1, FT, HIDDEN),
                         lambda b, f, be, ba, xb: (be[b], _feff(ba, b, f), 0)),
            pl.BlockSpec((1, FT, HIDDEN),
                         lambda b, f, be, ba, xb:
                         (be[b], NFT + _feff(ba, b, f), 0)),
            pl.BlockSpec((1, HIDDEN, FT),
                         lambda b, f, be, ba, xb: (be[b], 0, _feff(ba, b, f))),
        ],
        out_specs=pl.BlockSpec((RB, HIDDEN), lambda b, f, be, ba, xb: (b, 0)),
    )
    return pl.pallas_call(
        _gmm_body,
        grid_spec=grid_spec,
        out_shape=jax.ShapeDtypeStruct((CAP, HIDDEN), jnp.float32),
        compiler_params=pltpu.CompilerParams(
            dimension_semantics=("arbitrary", "arbitrary"),
        ),
    )(bev, bav, xbv, xs, rs, w13_16, w13_16, w2_16)


def _combine_body(pos0_hbm, pos1_hbm, y_hbm, out_hbm,
             i0_v, i1_v, r0_v, r1_v, s0, s1):
    c = lax.axis_index("c")
    s = lax.axis_index("s")
    w = s * NC + c
    tb = w * TCH
    pltpu.sync_copy(pos0_hbm.at[pl.ds(tb, TCH)], i0_v)
    pltpu.sync_copy(pos1_hbm.at[pl.ds(tb, TCH)], i1_v)

    def chunk_body(ci, carry):
        cp0 = pltpu.async_copy(
            y_hbm.at[i0_v.at[pl.ds(ci * ECH, ECH)]], r0_v, s0)
        cp1 = pltpu.async_copy(
            y_hbm.at[i1_v.at[pl.ds(ci * ECH, ECH)]], r1_v, s1)
        cp0.wait()
        cp1.wait()

        def row_body(i, carry2):
            def col_body(j, carry3):
                r0_v[i, pl.ds(j * 16, 16)] = (
                    r0_v[i, pl.ds(j * 16, 16)] + r1_v[i, pl.ds(j * 16, 16)])
                return carry3
            lax.fori_loop(0, HIDDEN // 16, col_body, 0)
            return carry2

        lax.fori_loop(0, ECH, row_body, 0)
        pltpu.sync_copy(r0_v, out_hbm.at[pl.ds(tb + ci * ECH, ECH)])
        return carry

    lax.fori_loop(0, TCH // ECH, chunk_body, 0)


_combine_impl = None


def _combine(pos0f, pos1f, y):
    global _combine_impl
    if _combine_impl is None:
        _combine_impl = pl.kernel(
            _combine_body,
            out_type=jax.ShapeDtypeStruct((T, HIDDEN), jnp.float32),
            mesh=_mesh(),
            scratch_types=[
                pltpu.VMEM((TCH,), jnp.int32),
                pltpu.VMEM((TCH,), jnp.int32),
                pltpu.VMEM((ECH, HIDDEN), jnp.float32),
                pltpu.VMEM((ECH, HIDDEN), jnp.float32),
                pltpu.SemaphoreType.DMA,
                pltpu.SemaphoreType.DMA,
            ],
            compiler_params=pltpu.CompilerParams(needs_layout_passes=False),
        )
    return _combine_impl(pos0f, pos1f, y)


def kernel(hidden_states, router_logits, w13_weight, w2_weight):
    w13_16 = w13_weight.astype(jnp.bfloat16)
    w2_16 = w2_weight.astype(jnp.bfloat16)

    pos0, pos1, w0, w1, be, ba, xb, xpack = _route(router_logits,
                                                   hidden_states)
    pos0f = pos0.reshape(T)
    pos1f = pos1.reshape(T)

    z32 = jnp.zeros((CAP,), jnp.int32)
    zf32 = jnp.zeros((CAP,), jnp.float32)
    xs, rs = _dispatch(pos0f, pos1f, w0.reshape(T), w1.reshape(T),
                       xpack, z32, zf32)
    y = _gmm(be.reshape(NB), ba.reshape(NB), xb.reshape(NB),
             xs, rs.reshape(CAP, 1), w13_16, w2_16)
    return _combine(pos0f, pos1f, y)


# speedup vs baseline: 2.0148x; 1.4290x over previous
import jax
import jax.numpy as jnp
from jax import lax
from jax.experimental import pallas as pl
from jax.experimental.pallas import tpu as pltpu
from jax.experimental.pallas import tpu_sc as plsc

NUM_EXPERTS = 8
HIDDEN = 1024
INTER = 2048
T = 2048

RB = 512
NB = 15
CAP = NB * RB
FT = 512
NFT = INTER // FT

NC = 2
NS = 16
NW = NC * NS
TCH = T // NW
ECH = 16


def _mesh():
    return plsc.VectorSubcoreMesh(core_axis_name="c", subcore_axis_name="s",
                                  num_cores=NC, num_subcores=NS)


def _route_body(logits_ref, pos0_ref, pos1_ref, wp0_ref, wp1_ref,
                be_ref, ba_ref):
    E = NUM_EXPERTS
    logits = logits_ref[...].astype(jnp.float32)
    iota_e = lax.broadcasted_iota(jnp.int32, (T, E), 1)
    m0 = jnp.max(logits, axis=1, keepdims=True)
    idx0 = jnp.min(jnp.where(logits == m0, iota_e, E), axis=1, keepdims=True)
    masked = jnp.where(iota_e == idx0, -jnp.inf, logits)
    m1 = jnp.max(masked, axis=1, keepdims=True)
    idx1 = jnp.min(jnp.where(masked == m1, iota_e, E), axis=1, keepdims=True)
    w0 = 1.0 / (1.0 + jnp.exp(m1 - m0))
    w1 = 1.0 - w0
    oh0 = (iota_e == idx0).astype(jnp.float32)
    oh1 = (iota_e == idx1).astype(jnp.float32)

    def excl_cumsum(a):
        s = a
        sh = 1
        while sh < T:
            s = s + jnp.concatenate(
                [jnp.zeros((sh, E), jnp.float32), s[:T - sh]], axis=0)
            sh *= 2
        return s - a

    c0 = excl_cumsum(oh0)
    c1 = excl_cumsum(oh1)
    count0 = jnp.sum(oh0, axis=0, keepdims=True)
    count = count0 + jnp.sum(oh1, axis=0, keepdims=True)
    pc = jnp.ceil(count / RB) * RB
    tri = (lax.broadcasted_iota(jnp.int32, (E, E), 0)
           < lax.broadcasted_iota(jnp.int32, (E, E), 1)).astype(jnp.float32)
    offs = lax.dot_general(pc, tri, (((1,), (0,)), ((), ())),
                           preferred_element_type=jnp.float32)
    total_used = jnp.sum(pc)
    rank0 = jnp.sum(oh0 * c0, axis=1, keepdims=True)
    rank1 = (jnp.sum(oh1 * c1, axis=1, keepdims=True)
             + jnp.sum(oh1 * count0, axis=1, keepdims=True))
    pos0_ref[...] = (jnp.sum(oh0 * offs, axis=1, keepdims=True)
                     + rank0).astype(jnp.int32)
    pos1_ref[...] = (jnp.sum(oh1 * offs, axis=1, keepdims=True)
                     + rank1).astype(jnp.int32)

    w0h = w0.astype(jnp.bfloat16)
    w0l = (w0 - w0h.astype(jnp.float32)).astype(jnp.bfloat16)
    w1h = w1.astype(jnp.bfloat16)
    w1l = (w1 - w1h.astype(jnp.float32)).astype(jnp.bfloat16)
    wp0_ref[...] = jnp.concatenate([w0h, w0l], axis=1)
    wp1_ref[...] = jnp.concatenate([w1h, w1l], axis=1)

    iota_b = lax.broadcasted_iota(jnp.int32, (1, NB), 1)
    bb = (iota_b * RB).astype(jnp.float32)
    bbase = jnp.minimum(bb, total_used - 1.0)
    acc = jnp.zeros((1, NB), jnp.float32)
    for e in range(E):
        off_e = lax.slice(offs, (0, e), (1, e + 1))
        acc = acc + (bbase >= off_e).astype(jnp.float32)
    be_ref[...] = (acc - 1.0).astype(jnp.int32)
    ba_ref[...] = (bb < total_used).astype(jnp.int32)


def _route(router_logits):
    return pl.pallas_call(
        _route_body,
        grid=(1,),
        in_specs=[pl.BlockSpec((T, NUM_EXPERTS), lambda i: (0, 0))],
        out_specs=[
            pl.BlockSpec((T, 1), lambda i: (0, 0)),
            pl.BlockSpec((T, 1), lambda i: (0, 0)),
            pl.BlockSpec((T, 2), lambda i: (0, 0)),
            pl.BlockSpec((T, 2), lambda i: (0, 0)),
            pl.BlockSpec((1, NB), lambda i: (0, 0)),
            pl.BlockSpec((1, NB), lambda i: (0, 0)),
        ],
        out_shape=[
            jax.ShapeDtypeStruct((T, 1), jnp.int32),
            jax.ShapeDtypeStruct((T, 1), jnp.int32),
            jax.ShapeDtypeStruct((T, 2), jnp.bfloat16),
            jax.ShapeDtypeStruct((T, 2), jnp.bfloat16),
            jax.ShapeDtypeStruct((1, NB), jnp.int32),
            jax.ShapeDtypeStruct((1, NB), jnp.int32),
        ],
    )(router_logits)


def _gmm_body(be_ref, ba_ref,
              pos0_ref, pos1_ref, wp0_ref, wp1_ref, x_ref,
              w13g_ref, w13u_ref, w2_ref, y_ref, xsb_ref, rs_ref):
    b = pl.program_id(0)
    f = pl.program_id(1)

    @pl.when(ba_ref[b] == 1)
    def _():
        @pl.when(f == 0)
        def _():
            iota_rb = lax.broadcasted_iota(jnp.int32, (T, RB), 1) + b * RB
            pt0 = (pos0_ref[...] == iota_rb).astype(jnp.bfloat16)
            pt1 = (pos1_ref[...] == iota_rb).astype(jnp.bfloat16)
            pt = pt0 + pt1
            xsb = lax.dot_general(pt, x_ref[...], (((0,), (0,)), ((), ())),
                                  preferred_element_type=jnp.float32)
            xsb_ref[...] = xsb.astype(jnp.bfloat16)
            r0 = lax.dot_general(pt0, wp0_ref[...], (((0,), (0,)), ((), ())),
                                 preferred_element_type=jnp.float32)
            r1 = lax.dot_general(pt1, wp1_ref[...], (((0,), (0,)), ((), ())),
                                 preferred_element_type=jnp.float32)
            rs_ref[...] = jnp.sum(r0 + r1, axis=1, keepdims=True)

        x = xsb_ref[...]
        gate = lax.dot_general(x, w13g_ref[0], (((1,), (1,)), ((), ())),
                               preferred_element_type=jnp.float32)
        up = lax.dot_general(x, w13u_ref[0], (((1,), (1,)), ((), ())),
                             preferred_element_type=jnp.float32)
        act = (gate * jax.nn.sigmoid(gate) * up).astype(jnp.bfloat16)
        part = lax.dot_general(act, w2_ref[0], (((1,), (1,)), ((), ())),
                               preferred_element_type=jnp.float32)

        @pl.when(f == 0)
        def _():
            y_ref[...] = part

        @pl.when(f > 0)
        def _():
            y_ref[...] += part

        @pl.when(f == NFT - 1)
        def _():
            y_ref[...] *= rs_ref[...]


def _feff(ba_ref, b, f):
    return jnp.where(ba_ref[b] == 0, NFT - 1, f)


def _gmm(bev, bav, pos0, pos1, wp0, wp1, x16, w13_16, w2_16):
    grid_spec = pltpu.PrefetchScalarGridSpec(
        num_scalar_prefetch=2,
        grid=(NB, NFT),
        in_specs=[
            pl.BlockSpec((T, 1), lambda b, f, be, ba: (0, 0)),
            pl.BlockSpec((T, 1), lambda b, f, be, ba: (0, 0)),
            pl.BlockSpec((T, 2), lambda b, f, be, ba: (0, 0)),
            pl.BlockSpec((T, 2), lambda b, f, be, ba: (0, 0)),
            pl.BlockSpec((T, HIDDEN), lambda b, f, be, ba: (0, 0)),
            pl.BlockSpec((1, FT, HIDDEN),
                         lambda b, f, be, ba: (be[b], _feff(ba, b, f), 0)),
            pl.BlockSpec((1, FT, HIDDEN),
                         lambda b, f, be, ba:
                         (be[b], NFT + _feff(ba, b, f), 0)),
            pl.BlockSpec((1, HIDDEN, FT),
                         lambda b, f, be, ba: (be[b], 0, _feff(ba, b, f))),
        ],
        out_specs=pl.BlockSpec((RB, HIDDEN), lambda b, f, be, ba: (b, 0)),
        scratch_shapes=[
            pltpu.VMEM((RB, HIDDEN), jnp.bfloat16),
            pltpu.VMEM((RB, 1), jnp.float32),
        ],
    )
    return pl.pallas_call(
        _gmm_body,
        grid_spec=grid_spec,
        out_shape=jax.ShapeDtypeStruct((CAP, HIDDEN), jnp.float32),
        compiler_params=pltpu.CompilerParams(
            dimension_semantics=("arbitrary", "arbitrary"),
        ),
    )(bev, bav, pos0, pos1, wp0, wp1, x16, w13_16, w13_16, w2_16)


def _combine_body(pos0_hbm, pos1_hbm, y_hbm, out_hbm,
                  i0_v, i1_v, r0_v, r1_v, s0, s1):
    c = lax.axis_index("c")
    s = lax.axis_index("s")
    w = s * NC + c
    tb = w * TCH
    pltpu.sync_copy(pos0_hbm.at[pl.ds(tb, TCH)], i0_v)
    pltpu.sync_copy(pos1_hbm.at[pl.ds(tb, TCH)], i1_v)

    def chunk_body(ci, carry):
        cp0 = pltpu.async_copy(
            y_hbm.at[i0_v.at[pl.ds(ci * ECH, ECH)]], r0_v, s0)
        cp1 = pltpu.async_copy(
            y_hbm.at[i1_v.at[pl.ds(ci * ECH, ECH)]], r1_v, s1)
        cp0.wait()
        cp1.wait()

        def row_body(i, carry2):
            def col_body(j, carry3):
                r0_v[i, pl.ds(j * 16, 16)] = (
                    r0_v[i, pl.ds(j * 16, 16)] + r1_v[i, pl.ds(j * 16, 16)])
                return carry3
            lax.fori_loop(0, HIDDEN // 16, col_body, 0)
            return carry2

        lax.fori_loop(0, ECH, row_body, 0)
        pltpu.sync_copy(r0_v, out_hbm.at[pl.ds(tb + ci * ECH, ECH)])
        return carry

    lax.fori_loop(0, TCH // ECH, chunk_body, 0)


_combine_impl = None


def _combine(pos0f, pos1f, y):
    global _combine_impl
    if _combine_impl is None:
        _combine_impl = pl.kernel(
            _combine_body,
            out_type=jax.ShapeDtypeStruct((T, HIDDEN), jnp.float32),
            mesh=_mesh(),
            scratch_types=[
                pltpu.VMEM((TCH,), jnp.int32),
                pltpu.VMEM((TCH,), jnp.int32),
                pltpu.VMEM((ECH, HIDDEN), jnp.float32),
                pltpu.VMEM((ECH, HIDDEN), jnp.float32),
                pltpu.SemaphoreType.DMA,
                pltpu.SemaphoreType.DMA,
            ],
            compiler_params=pltpu.CompilerParams(needs_layout_passes=False),
        )
    return _combine_impl(pos0f, pos1f, y)


def kernel(hidden_states, router_logits, w13_weight, w2_weight):
    x16 = hidden_states.astype(jnp.bfloat16)
    w13_16 = w13_weight.astype(jnp.bfloat16)
    w2_16 = w2_weight.astype(jnp.bfloat16)

    pos0, pos1, wp0, wp1, be, ba = _route(router_logits)
    y = _gmm(be.reshape(NB), ba.reshape(NB), pos0, pos1, wp0, wp1,
             x16, w13_16, w2_16)
    return _combine(pos0.reshape(T), pos1.reshape(T), y)


# speedup vs baseline: 2.2470x; 1.1152x over previous
import jax
import jax.numpy as jnp
from jax import lax
from jax.experimental import pallas as pl
from jax.experimental.pallas import tpu as pltpu
from jax.experimental.pallas import tpu_sc as plsc

NUM_EXPERTS = 8
HIDDEN = 1024
INTER = 2048
T = 2048

RB = 512
NB = 15
CAP = NB * RB
FT = 512
NFT = INTER // FT

NC = 2
NS = 16
NW = NC * NS
TCH = T // NW
ECH = 16


def _mesh():
    return plsc.VectorSubcoreMesh(core_axis_name="c", subcore_axis_name="s",
                                  num_cores=NC, num_subcores=NS)


def _route_body(logits_ref, pos0_ref, pos1_ref, wp0_ref, wp1_ref,
                be_ref, ba_ref, xb_ref):
    E = NUM_EXPERTS
    logits = logits_ref[...].astype(jnp.float32)
    iota_e = lax.broadcasted_iota(jnp.int32, (T, E), 1)
    m0 = jnp.max(logits, axis=1, keepdims=True)
    idx0 = jnp.min(jnp.where(logits == m0, iota_e, E), axis=1, keepdims=True)
    masked = jnp.where(iota_e == idx0, -jnp.inf, logits)
    m1 = jnp.max(masked, axis=1, keepdims=True)
    idx1 = jnp.min(jnp.where(masked == m1, iota_e, E), axis=1, keepdims=True)
    w0 = 1.0 / (1.0 + jnp.exp(m1 - m0))
    w1 = 1.0 - w0
    oh0 = (iota_e == idx0).astype(jnp.float32)
    oh1 = (iota_e == idx1).astype(jnp.float32)

    def excl_cumsum(a):
        s = a
        sh = 1
        while sh < T:
            s = s + jnp.concatenate(
                [jnp.zeros((sh, E), jnp.float32), s[:T - sh]], axis=0)
            sh *= 2
        return s - a

    c0 = excl_cumsum(oh0)
    c1 = excl_cumsum(oh1)
    count0 = jnp.sum(oh0, axis=0, keepdims=True)
    count = count0 + jnp.sum(oh1, axis=0, keepdims=True)
    pc = jnp.ceil(count / RB) * RB
    tri = (lax.broadcasted_iota(jnp.int32, (E, E), 0)
           < lax.broadcasted_iota(jnp.int32, (E, E), 1)).astype(jnp.float32)
    offs = lax.dot_general(pc, tri, (((1,), (0,)), ((), ())),
                           preferred_element_type=jnp.float32)
    total_used = jnp.sum(pc)
    rank0 = jnp.sum(oh0 * c0, axis=1, keepdims=True)
    rank1 = (jnp.sum(oh1 * c1, axis=1, keepdims=True)
             + jnp.sum(oh1 * count0, axis=1, keepdims=True))
    pos0_ref[...] = (jnp.sum(oh0 * offs, axis=1, keepdims=True)
                     + rank0).astype(jnp.int32)
    pos1_ref[...] = (jnp.sum(oh1 * offs, axis=1, keepdims=True)
                     + rank1).astype(jnp.int32)

    w0h = w0.astype(jnp.bfloat16)
    w0l = (w0 - w0h.astype(jnp.float32)).astype(jnp.bfloat16)
    w1h = w1.astype(jnp.bfloat16)
    w1l = (w1 - w1h.astype(jnp.float32)).astype(jnp.bfloat16)
    wp0_ref[...] = jnp.concatenate([w0h, w0l], axis=1)
    wp1_ref[...] = jnp.concatenate([w1h, w1l], axis=1)

    iota_b = lax.broadcasted_iota(jnp.int32, (1, NB), 1)
    bb = (iota_b * RB).astype(jnp.float32)
    bbase = jnp.minimum(bb, total_used - 1.0)
    acc = jnp.zeros((1, NB), jnp.float32)
    for e in range(E):
        off_e = lax.slice(offs, (0, e), (1, e + 1))
        acc = acc + (bbase >= off_e).astype(jnp.float32)
    be_ref[...] = (acc - 1.0).astype(jnp.int32)
    ba_ref[...] = (bb < total_used).astype(jnp.int32)
    nbt = (total_used / RB).astype(jnp.int32)
    xb_ref[...] = jnp.minimum(iota_b, nbt - 1)


def _route(router_logits):
    return pl.pallas_call(
        _route_body,
        grid=(1,),
        in_specs=[pl.BlockSpec((T, NUM_EXPERTS), lambda i: (0, 0))],
        out_specs=[
            pl.BlockSpec((T, 1), lambda i: (0, 0)),
            pl.BlockSpec((T, 1), lambda i: (0, 0)),
            pl.BlockSpec((T, 2), lambda i: (0, 0)),
            pl.BlockSpec((T, 2), lambda i: (0, 0)),
            pl.BlockSpec((1, NB), lambda i: (0, 0)),
            pl.BlockSpec((1, NB), lambda i: (0, 0)),
            pl.BlockSpec((1, NB), lambda i: (0, 0)),
        ],
        out_shape=[
            jax.ShapeDtypeStruct((T, 1), jnp.int32),
            jax.ShapeDtypeStruct((T, 1), jnp.int32),
            jax.ShapeDtypeStruct((T, 2), jnp.bfloat16),
            jax.ShapeDtypeStruct((T, 2), jnp.bfloat16),
            jax.ShapeDtypeStruct((1, NB), jnp.int32),
            jax.ShapeDtypeStruct((1, NB), jnp.int32),
            jax.ShapeDtypeStruct((1, NB), jnp.int32),
        ],
    )(router_logits)


def _gmm_body(be_ref, ba_ref, xb_ref,
              pos0_ref, pos1_ref, wp0_ref, wp1_ref, x_ref,
              w13g_ref, w13u_ref, w2_ref, y_ref):
    b = pl.program_id(0)

    @pl.when(ba_ref[b] == 1)
    def _():
        iota_rb = lax.broadcasted_iota(jnp.int32, (T, RB), 1) + b * RB
        pt0 = (pos0_ref[...] == iota_rb).astype(jnp.bfloat16)
        pt1 = (pos1_ref[...] == iota_rb).astype(jnp.bfloat16)
        pt = pt0 + pt1
        xsb = lax.dot_general(pt, x_ref[...], (((0,), (0,)), ((), ())),
                              preferred_element_type=jnp.float32
                              ).astype(jnp.bfloat16)
        r0 = lax.dot_general(pt0, wp0_ref[...], (((0,), (0,)), ((), ())),
                             preferred_element_type=jnp.float32)
        r1 = lax.dot_general(pt1, wp1_ref[...], (((0,), (0,)), ((), ())),
                             preferred_element_type=jnp.float32)
        rs = jnp.sum(r0 + r1, axis=1, keepdims=True)

        acc = jnp.zeros((RB, HIDDEN), jnp.float32)
        for fh in range(NFT):
            wg = w13g_ref[0, pl.ds(fh * FT, FT), :]
            wu = w13u_ref[0, pl.ds(fh * FT, FT), :]
            gate = lax.dot_general(xsb, wg, (((1,), (1,)), ((), ())),
                                   preferred_element_type=jnp.float32)
            up = lax.dot_general(xsb, wu, (((1,), (1,)), ((), ())),
                                 preferred_element_type=jnp.float32)
            act = (gate * jax.nn.sigmoid(gate) * up).astype(jnp.bfloat16)
            w2t = w2_ref[0, :, pl.ds(fh * FT, FT)]
            acc = acc + lax.dot_general(act, w2t, (((1,), (1,)), ((), ())),
                                        preferred_element_type=jnp.float32)
        y_ref[...] = acc * rs


def _gmm(bev, bav, xbv, pos0, pos1, wp0, wp1, x16, w13_16, w2_16):
    grid_spec = pltpu.PrefetchScalarGridSpec(
        num_scalar_prefetch=3,
        grid=(NB,),
        in_specs=[
            pl.BlockSpec((T, 1), lambda b, be, ba, xb: (0, 0)),
            pl.BlockSpec((T, 1), lambda b, be, ba, xb: (0, 0)),
            pl.BlockSpec((T, 2), lambda b, be, ba, xb: (0, 0)),
            pl.BlockSpec((T, 2), lambda b, be, ba, xb: (0, 0)),
            pl.BlockSpec((T, HIDDEN), lambda b, be, ba, xb: (0, 0)),
            pl.BlockSpec((1, INTER, HIDDEN),
                         lambda b, be, ba, xb: (be[b], 0, 0)),
            pl.BlockSpec((1, INTER, HIDDEN),
                         lambda b, be, ba, xb: (be[b], 1, 0)),
            pl.BlockSpec((1, HIDDEN, INTER),
                         lambda b, be, ba, xb: (be[b], 0, 0)),
        ],
        out_specs=pl.BlockSpec((RB, HIDDEN), lambda b, be, ba, xb: (xb[b], 0)),
    )
    return pl.pallas_call(
        _gmm_body,
        grid_spec=grid_spec,
        out_shape=jax.ShapeDtypeStruct((CAP, HIDDEN), jnp.float32),
        compiler_params=pltpu.CompilerParams(
            dimension_semantics=("arbitrary",),
        ),
    )(bev, bav, xbv, pos0, pos1, wp0, wp1, x16, w13_16, w13_16, w2_16)


def _combine_body(pos0_hbm, pos1_hbm, y_hbm, out_hbm,
                  i0_v, i1_v, r0_v, r1_v, s0, s1):
    c = lax.axis_index("c")
    s = lax.axis_index("s")
    w = s * NC + c
    tb = w * TCH
    pltpu.sync_copy(pos0_hbm.at[pl.ds(tb, TCH)], i0_v)
    pltpu.sync_copy(pos1_hbm.at[pl.ds(tb, TCH)], i1_v)

    def chunk_body(ci, carry):
        cp0 = pltpu.async_copy(
            y_hbm.at[i0_v.at[pl.ds(ci * ECH, ECH)]], r0_v, s0)
        cp1 = pltpu.async_copy(
            y_hbm.at[i1_v.at[pl.ds(ci * ECH, ECH)]], r1_v, s1)
        cp0.wait()
        cp1.wait()

        def row_body(i, carry2):
            def col_body(j, carry3):
                r0_v[i, pl.ds(j * 16, 16)] = (
                    r0_v[i, pl.ds(j * 16, 16)] + r1_v[i, pl.ds(j * 16, 16)])
                return carry3
            lax.fori_loop(0, HIDDEN // 16, col_body, 0)
            return carry2

        lax.fori_loop(0, ECH, row_body, 0)
        pltpu.sync_copy(r0_v, out_hbm.at[pl.ds(tb + ci * ECH, ECH)])
        return carry

    lax.fori_loop(0, TCH // ECH, chunk_body, 0)


_combine_impl = None


def _combine(pos0f, pos1f, y):
    global _combine_impl
    if _combine_impl is None:
        _combine_impl = pl.kernel(
            _combine_body,
            out_type=jax.ShapeDtypeStruct((T, HIDDEN), jnp.float32),
            mesh=_mesh(),
            scratch_types=[
                pltpu.VMEM((TCH,), jnp.int32),
                pltpu.VMEM((TCH,), jnp.int32),
                pltpu.VMEM((ECH, HIDDEN), jnp.float32),
                pltpu.VMEM((ECH, HIDDEN), jnp.float32),
                pltpu.SemaphoreType.DMA,
                pltpu.SemaphoreType.DMA,
            ],
            compiler_params=pltpu.CompilerParams(needs_layout_passes=False),
        )
    return _combine_impl(pos0f, pos1f, y)


def kernel(hidden_states, router_logits, w13_weight, w2_weight):
    x16 = hidden_states.astype(jnp.bfloat16)
    w13_16 = w13_weight.astype(jnp.bfloat16)
    w2_16 = w2_weight.astype(jnp.bfloat16)

    pos0, pos1, wp0, wp1, be, ba, xb = _route(router_logits)
    y = _gmm(be.reshape(NB), ba.reshape(NB), xb.reshape(NB),
             pos0, pos1, wp0, wp1, x16, w13_16, w2_16)
    return _combine(pos0.reshape(T), pos1.reshape(T), y)


# speedup vs baseline: 2.8955x; 1.2886x over previous
import jax
import jax.numpy as jnp
from jax import lax
from jax.experimental import pallas as pl
from jax.experimental.pallas import tpu as pltpu
from jax.experimental.pallas import tpu_sc as plsc

NUM_EXPERTS = 8
HIDDEN = 1024
INTER = 2048
T = 2048

RB = 512
NB = 15
CAP = NB * RB
FT = 512
NFT = INTER // FT

NC = 2
NS = 16
NW = NC * NS
TCH = T // NW
ECH = 16


def _mesh():
    return plsc.VectorSubcoreMesh(core_axis_name="c", subcore_axis_name="s",
                                  num_cores=NC, num_subcores=NS)


def _route_body(logits_ref, pos0_ref, pos1_ref, wp0_ref, wp1_ref,
                be_ref, ba_ref, xb_ref):
    E = NUM_EXPERTS
    logits = logits_ref[...].astype(jnp.float32)
    iota_e = lax.broadcasted_iota(jnp.int32, (T, E), 1)
    m0 = jnp.max(logits, axis=1, keepdims=True)
    idx0 = jnp.min(jnp.where(logits == m0, iota_e, E), axis=1, keepdims=True)
    masked = jnp.where(iota_e == idx0, -jnp.inf, logits)
    m1 = jnp.max(masked, axis=1, keepdims=True)
    idx1 = jnp.min(jnp.where(masked == m1, iota_e, E), axis=1, keepdims=True)
    w0 = 1.0 / (1.0 + jnp.exp(m1 - m0))
    w1 = 1.0 - w0
    oh0 = (iota_e == idx0).astype(jnp.float32)
    oh1 = (iota_e == idx1).astype(jnp.float32)

    def excl_cumsum(a):
        s = a
        sh = 1
        while sh < T:
            s = s + jnp.concatenate(
                [jnp.zeros((sh, E), jnp.float32), s[:T - sh]], axis=0)
            sh *= 2
        return s - a

    c0 = excl_cumsum(oh0)
    c1 = excl_cumsum(oh1)
    count0 = jnp.sum(oh0, axis=0, keepdims=True)
    count = count0 + jnp.sum(oh1, axis=0, keepdims=True)
    pc = jnp.ceil(count / RB) * RB
    tri = (lax.broadcasted_iota(jnp.int32, (E, E), 0)
           < lax.broadcasted_iota(jnp.int32, (E, E), 1)).astype(jnp.float32)
    offs = lax.dot_general(pc, tri, (((1,), (0,)), ((), ())),
                           preferred_element_type=jnp.float32)
    total_used = jnp.sum(pc)
    rank0 = jnp.sum(oh0 * c0, axis=1, keepdims=True)
    rank1 = (jnp.sum(oh1 * c1, axis=1, keepdims=True)
             + jnp.sum(oh1 * count0, axis=1, keepdims=True))
    pos0_ref[...] = (jnp.sum(oh0 * offs, axis=1, keepdims=True)
                     + rank0).astype(jnp.int32)
    pos1_ref[...] = (jnp.sum(oh1 * offs, axis=1, keepdims=True)
                     + rank1).astype(jnp.int32)

    w0h = w0.astype(jnp.bfloat16)
    w0l = (w0 - w0h.astype(jnp.float32)).astype(jnp.bfloat16)
    w1h = w1.astype(jnp.bfloat16)
    w1l = (w1 - w1h.astype(jnp.float32)).astype(jnp.bfloat16)
    wp0_ref[...] = jnp.concatenate([w0h, w0l], axis=1)
    wp1_ref[...] = jnp.concatenate([w1h, w1l], axis=1)

    iota_b = lax.broadcasted_iota(jnp.int32, (1, NB), 1)
    bb = (iota_b * RB).astype(jnp.float32)
    bbase = jnp.minimum(bb, total_used - 1.0)
    acc = jnp.zeros((1, NB), jnp.float32)
    for e in range(E):
        off_e = lax.slice(offs, (0, e), (1, e + 1))
        acc = acc + (bbase >= off_e).astype(jnp.float32)
    be_ref[...] = (acc - 1.0).astype(jnp.int32)
    ba_ref[...] = (bb < total_used).astype(jnp.int32)
    nbt = (total_used / RB).astype(jnp.int32)
    xb_ref[...] = jnp.minimum(iota_b, nbt - 1)


def _route(router_logits):
    return pl.pallas_call(
        _route_body,
        grid=(1,),
        in_specs=[pl.BlockSpec((T, NUM_EXPERTS), lambda i: (0, 0))],
        out_specs=[
            pl.BlockSpec((T, 1), lambda i: (0, 0)),
            pl.BlockSpec((T, 1), lambda i: (0, 0)),
            pl.BlockSpec((T, 2), lambda i: (0, 0)),
            pl.BlockSpec((T, 2), lambda i: (0, 0)),
            pl.BlockSpec((1, NB), lambda i: (0, 0)),
            pl.BlockSpec((1, NB), lambda i: (0, 0)),
            pl.BlockSpec((1, NB), lambda i: (0, 0)),
        ],
        out_shape=[
            jax.ShapeDtypeStruct((T, 1), jnp.int32),
            jax.ShapeDtypeStruct((T, 1), jnp.int32),
            jax.ShapeDtypeStruct((T, 2), jnp.bfloat16),
            jax.ShapeDtypeStruct((T, 2), jnp.bfloat16),
            jax.ShapeDtypeStruct((1, NB), jnp.int32),
            jax.ShapeDtypeStruct((1, NB), jnp.int32),
            jax.ShapeDtypeStruct((1, NB), jnp.int32),
        ],
    )(router_logits)


def _gmm_body(be_ref, ba_ref, xb_ref,
              pos0_ref, pos1_ref, wp0_ref, wp1_ref, x_ref,
              w13g_ref, w13u_ref, w2_ref, y_ref, xsb_ref, rs_ref):
    b = pl.program_id(0)
    h = pl.program_id(1)

    @pl.when(ba_ref[b] == 1)
    def _():
        @pl.when(h == 0)
        def _():
            iota_rb = lax.broadcasted_iota(jnp.int32, (T, RB), 1) + b * RB
            pt0 = (pos0_ref[...] == iota_rb).astype(jnp.bfloat16)
            pt1 = (pos1_ref[...] == iota_rb).astype(jnp.bfloat16)
            pt = pt0 + pt1
            xsb_ref[...] = lax.dot_general(
                pt, x_ref[...], (((0,), (0,)), ((), ())),
                preferred_element_type=jnp.float32).astype(jnp.bfloat16)
            r0 = lax.dot_general(pt0, wp0_ref[...], (((0,), (0,)), ((), ())),
                                 preferred_element_type=jnp.float32)
            r1 = lax.dot_general(pt1, wp1_ref[...], (((0,), (0,)), ((), ())),
                                 preferred_element_type=jnp.float32)
            rs_ref[...] = jnp.sum(r0 + r1, axis=1, keepdims=True)

        xsb = xsb_ref[...]
        wg = w13g_ref[0].astype(jnp.bfloat16)
        wu = w13u_ref[0].astype(jnp.bfloat16)
        w2t = w2_ref[0].astype(jnp.bfloat16)
        gate = lax.dot_general(xsb, wg, (((1,), (1,)), ((), ())),
                               preferred_element_type=jnp.float32)
        up = lax.dot_general(xsb, wu, (((1,), (1,)), ((), ())),
                             preferred_element_type=jnp.float32)
        act = (gate * jax.nn.sigmoid(gate) * up).astype(jnp.bfloat16)
        part = lax.dot_general(act, w2t, (((1,), (1,)), ((), ())),
                               preferred_element_type=jnp.float32)

        @pl.when(h == 0)
        def _():
            y_ref[...] = part

        @pl.when(h == 1)
        def _():
            y_ref[...] = (y_ref[...] + part) * rs_ref[...]


def _heff(ba_ref, b, h):
    return jnp.where(ba_ref[b] == 0, 1, h)


def _gmm(bev, bav, xbv, pos0, pos1, wp0, wp1, x16, w13, w2):
    grid_spec = pltpu.PrefetchScalarGridSpec(
        num_scalar_prefetch=3,
        grid=(NB, 2),
        in_specs=[
            pl.BlockSpec((T, 1), lambda b, h, be, ba, xb: (0, 0)),
            pl.BlockSpec((T, 1), lambda b, h, be, ba, xb: (0, 0)),
            pl.BlockSpec((T, 2), lambda b, h, be, ba, xb: (0, 0)),
            pl.BlockSpec((T, 2), lambda b, h, be, ba, xb: (0, 0)),
            pl.BlockSpec((T, HIDDEN), lambda b, h, be, ba, xb: (0, 0)),
            pl.BlockSpec((1, INTER // 2, HIDDEN),
                         lambda b, h, be, ba, xb:
                         (be[b], _heff(ba, b, h), 0)),
            pl.BlockSpec((1, INTER // 2, HIDDEN),
                         lambda b, h, be, ba, xb:
                         (be[b], 2 + _heff(ba, b, h), 0)),
            pl.BlockSpec((1, HIDDEN, INTER // 2),
                         lambda b, h, be, ba, xb:
                         (be[b], 0, _heff(ba, b, h))),
        ],
        out_specs=pl.BlockSpec((RB, HIDDEN),
                               lambda b, h, be, ba, xb: (xb[b], 0)),
        scratch_shapes=[
            pltpu.VMEM((RB, HIDDEN), jnp.bfloat16),
            pltpu.VMEM((RB, 1), jnp.float32),
        ],
    )
    return pl.pallas_call(
        _gmm_body,
        grid_spec=grid_spec,
        out_shape=jax.ShapeDtypeStruct((CAP, HIDDEN), jnp.float32),
        compiler_params=pltpu.CompilerParams(
            dimension_semantics=("arbitrary", "arbitrary"),
        ),
    )(bev, bav, xbv, pos0, pos1, wp0, wp1, x16, w13, w13, w2)


def _combine_body(pos0_hbm, pos1_hbm, y_hbm, out_hbm,
                  i0_v, i1_v, r0_v, r1_v, s0, s1):
    c = lax.axis_index("c")
    s = lax.axis_index("s")
    w = s * NC + c
    tb = w * TCH
    pltpu.sync_copy(pos0_hbm.at[pl.ds(tb, TCH)], i0_v)
    pltpu.sync_copy(pos1_hbm.at[pl.ds(tb, TCH)], i1_v)

    def chunk_body(ci, carry):
        cp0 = pltpu.async_copy(
            y_hbm.at[i0_v.at[pl.ds(ci * ECH, ECH)]], r0_v, s0)
        cp1 = pltpu.async_copy(
            y_hbm.at[i1_v.at[pl.ds(ci * ECH, ECH)]], r1_v, s1)
        cp0.wait()
        cp1.wait()

        def row_body(i, carry2):
            def col_body(j, carry3):
                r0_v[i, pl.ds(j * 16, 16)] = (
                    r0_v[i, pl.ds(j * 16, 16)] + r1_v[i, pl.ds(j * 16, 16)])
                return carry3
            lax.fori_loop(0, HIDDEN // 16, col_body, 0)
            return carry2

        lax.fori_loop(0, ECH, row_body, 0)
        pltpu.sync_copy(r0_v, out_hbm.at[pl.ds(tb + ci * ECH, ECH)])
        return carry

    lax.fori_loop(0, TCH // ECH, chunk_body, 0)


_combine_impl = None


def _combine(pos0f, pos1f, y):
    global _combine_impl
    if _combine_impl is None:
        _combine_impl = pl.kernel(
            _combine_body,
            out_type=jax.ShapeDtypeStruct((T, HIDDEN), jnp.float32),
            mesh=_mesh(),
            scratch_types=[
                pltpu.VMEM((TCH,), jnp.int32),
                pltpu.VMEM((TCH,), jnp.int32),
                pltpu.VMEM((ECH, HIDDEN), jnp.float32),
                pltpu.VMEM((ECH, HIDDEN), jnp.float32),
                pltpu.SemaphoreType.DMA,
                pltpu.SemaphoreType.DMA,
            ],
            compiler_params=pltpu.CompilerParams(needs_layout_passes=False),
        )
    return _combine_impl(pos0f, pos1f, y)


def kernel(hidden_states, router_logits, w13_weight, w2_weight):
    x16 = hidden_states.astype(jnp.bfloat16)

    pos0, pos1, wp0, wp1, be, ba, xb = _route(router_logits)
    y = _gmm(be.reshape(NB), ba.reshape(NB), xb.reshape(NB),
             pos0, pos1, wp0, wp1, x16, w13_weight, w2_weight)
    return _combine(pos0.reshape(T), pos1.reshape(T), y)


# speedup vs baseline: 2.8957x; 1.0001x over previous
import jax
import jax.numpy as jnp
from jax import lax
from jax.experimental import pallas as pl
from jax.experimental.pallas import tpu as pltpu
from jax.experimental.pallas import tpu_sc as plsc

NUM_EXPERTS = 8
HIDDEN = 1024
INTER = 2048
T = 2048

RB = 512
NB = 15
CAP = NB * RB
NC = 2
NS = 16
NW = NC * NS
TCH = T // NW
ECH = 16


def _mesh():
    return plsc.VectorSubcoreMesh(core_axis_name="c", subcore_axis_name="s",
                                  num_cores=NC, num_subcores=NS)


def _route_body(logits_ref, pos0_ref, pos1_ref, wp0_ref, wp1_ref,
                be_ref, ba_ref, xb_ref):
    E = NUM_EXPERTS
    logits = logits_ref[...].astype(jnp.float32)
    iota_e = lax.broadcasted_iota(jnp.int32, (T, E), 1)
    m0 = jnp.max(logits, axis=1, keepdims=True)
    idx0 = jnp.min(jnp.where(logits == m0, iota_e, E), axis=1, keepdims=True)
    masked = jnp.where(iota_e == idx0, -jnp.inf, logits)
    m1 = jnp.max(masked, axis=1, keepdims=True)
    idx1 = jnp.min(jnp.where(masked == m1, iota_e, E), axis=1, keepdims=True)
    w0 = 1.0 / (1.0 + jnp.exp(m1 - m0))
    w1 = 1.0 - w0
    oh0 = (iota_e == idx0).astype(jnp.float32)
    oh1 = (iota_e == idx1).astype(jnp.float32)

    def excl_cumsum(a):
        s = a
        sh = 1
        while sh < T:
            s = s + jnp.concatenate(
                [jnp.zeros((sh, E), jnp.float32), s[:T - sh]], axis=0)
            sh *= 2
        return s - a

    c0 = excl_cumsum(oh0)
    c1 = excl_cumsum(oh1)
    count0 = jnp.sum(oh0, axis=0, keepdims=True)
    count = count0 + jnp.sum(oh1, axis=0, keepdims=True)
    pc = jnp.ceil(count / RB) * RB
    tri = (lax.broadcasted_iota(jnp.int32, (E, E), 0)
           < lax.broadcasted_iota(jnp.int32, (E, E), 1)).astype(jnp.float32)
    offs = lax.dot_general(pc, tri, (((1,), (0,)), ((), ())),
                           preferred_element_type=jnp.float32)
    total_used = jnp.sum(pc)
    rank0 = jnp.sum(oh0 * c0, axis=1, keepdims=True)
    rank1 = (jnp.sum(oh1 * c1, axis=1, keepdims=True)
             + jnp.sum(oh1 * count0, axis=1, keepdims=True))
    pos0_ref[...] = (jnp.sum(oh0 * offs, axis=1, keepdims=True)
                     + rank0).astype(jnp.int32)
    pos1_ref[...] = (jnp.sum(oh1 * offs, axis=1, keepdims=True)
                     + rank1).astype(jnp.int32)

    w0h = w0.astype(jnp.bfloat16)
    w0l = (w0 - w0h.astype(jnp.float32)).astype(jnp.bfloat16)
    w1h = w1.astype(jnp.bfloat16)
    w1l = (w1 - w1h.astype(jnp.float32)).astype(jnp.bfloat16)
    wp0_ref[...] = jnp.concatenate([w0h, w0l], axis=1)
    wp1_ref[...] = jnp.concatenate([w1h, w1l], axis=1)

    iota_b = lax.broadcasted_iota(jnp.int32, (1, NB), 1)
    bb = (iota_b * RB).astype(jnp.float32)
    bbase = jnp.minimum(bb, total_used - 1.0)
    acc = jnp.zeros((1, NB), jnp.float32)
    for e in range(E):
        off_e = lax.slice(offs, (0, e), (1, e + 1))
        acc = acc + (bbase >= off_e).astype(jnp.float32)
    be_ref[...] = (acc - 1.0).astype(jnp.int32)
    ba_ref[...] = (bb < total_used).astype(jnp.int32)
    nbt = (total_used / RB).astype(jnp.int32)
    xb_ref[...] = jnp.minimum(iota_b, nbt - 1)


def _route(router_logits):
    return pl.pallas_call(
        _route_body,
        grid=(1,),
        in_specs=[pl.BlockSpec((T, NUM_EXPERTS), lambda i: (0, 0))],
        out_specs=[
            pl.BlockSpec((T, 1), lambda i: (0, 0)),
            pl.BlockSpec((T, 1), lambda i: (0, 0)),
            pl.BlockSpec((T, 2), lambda i: (0, 0)),
            pl.BlockSpec((T, 2), lambda i: (0, 0)),
            pl.BlockSpec((1, NB), lambda i: (0, 0)),
            pl.BlockSpec((1, NB), lambda i: (0, 0)),
            pl.BlockSpec((1, NB), lambda i: (0, 0)),
        ],
        out_shape=[
            jax.ShapeDtypeStruct((T, 1), jnp.int32),
            jax.ShapeDtypeStruct((T, 1), jnp.int32),
            jax.ShapeDtypeStruct((T, 2), jnp.bfloat16),
            jax.ShapeDtypeStruct((T, 2), jnp.bfloat16),
            jax.ShapeDtypeStruct((1, NB), jnp.int32),
            jax.ShapeDtypeStruct((1, NB), jnp.int32),
            jax.ShapeDtypeStruct((1, NB), jnp.int32),
        ],
    )(router_logits)


def _gmm_body(be_ref, ba_ref, xb_ref,
              pos0_ref, pos1_ref, wp0_ref, wp1_ref, x_ref,
              w13g_ref, w13u_ref, w2_ref, y_ref, xsb_ref, rs_ref):
    b = pl.program_id(0)
    h = pl.program_id(1)

    @pl.when(ba_ref[b] == 1)
    def _():
        @pl.when(h == 0)
        def _():
            iota_rb = lax.broadcasted_iota(jnp.int32, (T, RB), 1) + b * RB
            pt0 = (pos0_ref[...] == iota_rb).astype(jnp.bfloat16)
            pt1 = (pos1_ref[...] == iota_rb).astype(jnp.bfloat16)
            pt = pt0 + pt1
            xsb_ref[...] = lax.dot_general(
                pt, x_ref[...], (((0,), (0,)), ((), ())),
                preferred_element_type=jnp.float32).astype(jnp.bfloat16)
            r0 = lax.dot_general(pt0, wp0_ref[...], (((0,), (0,)), ((), ())),
                                 preferred_element_type=jnp.float32)
            r1 = lax.dot_general(pt1, wp1_ref[...], (((0,), (0,)), ((), ())),
                                 preferred_element_type=jnp.float32)
            rs_ref[...] = jnp.sum(r0 + r1, axis=1, keepdims=True)

        xsb = xsb_ref[...]
        wg = w13g_ref[0].astype(jnp.bfloat16)
        wu = w13u_ref[0].astype(jnp.bfloat16)
        w2t = w2_ref[0].astype(jnp.bfloat16)
        gate = lax.dot_general(xsb, wg, (((1,), (1,)), ((), ())),
                               preferred_element_type=jnp.float32)
        up = lax.dot_general(xsb, wu, (((1,), (1,)), ((), ())),
                             preferred_element_type=jnp.float32)
        act = (gate * jax.nn.sigmoid(gate) * up).astype(jnp.bfloat16)
        part = lax.dot_general(act, w2t, (((1,), (1,)), ((), ())),
                               preferred_element_type=jnp.float32)

        @pl.when(h == 0)
        def _():
            y_ref[...] = part

        @pl.when(h == 1)
        def _():
            y_ref[...] = (y_ref[...] + part) * rs_ref[...]


def _heff(ba_ref, b, h):
    return jnp.where(ba_ref[b] == 0, 1, h)


def _gmm(bev, bav, xbv, pos0, pos1, wp0, wp1, x16, w13, w2):
    grid_spec = pltpu.PrefetchScalarGridSpec(
        num_scalar_prefetch=3,
        grid=(NB, 2),
        in_specs=[
            pl.BlockSpec((T, 1), lambda b, h, be, ba, xb: (0, 0)),
            pl.BlockSpec((T, 1), lambda b, h, be, ba, xb: (0, 0)),
            pl.BlockSpec((T, 2), lambda b, h, be, ba, xb: (0, 0)),
            pl.BlockSpec((T, 2), lambda b, h, be, ba, xb: (0, 0)),
            pl.BlockSpec((T, HIDDEN), lambda b, h, be, ba, xb: (0, 0)),
            pl.BlockSpec((1, INTER // 2, HIDDEN),
                         lambda b, h, be, ba, xb:
                         (be[b], _heff(ba, b, h), 0)),
            pl.BlockSpec((1, INTER // 2, HIDDEN),
                         lambda b, h, be, ba, xb:
                         (be[b], 2 + _heff(ba, b, h), 0)),
            pl.BlockSpec((1, HIDDEN, INTER // 2),
                         lambda b, h, be, ba, xb:
                         (be[b], 0, _heff(ba, b, h))),
        ],
        out_specs=pl.BlockSpec((RB, HIDDEN),
                               lambda b, h, be, ba, xb: (xb[b], 0)),
        scratch_shapes=[
            pltpu.VMEM((RB, HIDDEN), jnp.bfloat16),
            pltpu.VMEM((RB, 1), jnp.float32),
        ],
    )
    return pl.pallas_call(
        _gmm_body,
        grid_spec=grid_spec,
        out_shape=jax.ShapeDtypeStruct((CAP, HIDDEN), jnp.float32),
        compiler_params=pltpu.CompilerParams(
            dimension_semantics=("arbitrary", "arbitrary"),
        ),
    )(bev, bav, xbv, pos0, pos1, wp0, wp1, x16, w13, w13, w2)


def _combine_body(pos0_hbm, pos1_hbm, y_hbm, out_hbm,
                  i0_v, i1_v, r0_v, r1_v, s0, s1):
    c = lax.axis_index("c")
    s = lax.axis_index("s")
    w = s * NC + c
    tb = w * TCH
    pltpu.sync_copy(pos0_hbm.at[pl.ds(tb, TCH)], i0_v)
    pltpu.sync_copy(pos1_hbm.at[pl.ds(tb, TCH)], i1_v)

    def chunk_body(ci, carry):
        cp0 = pltpu.async_copy(
            y_hbm.at[i0_v.at[pl.ds(ci * ECH, ECH)]], r0_v, s0)
        cp1 = pltpu.async_copy(
            y_hbm.at[i1_v.at[pl.ds(ci * ECH, ECH)]], r1_v, s1)
        cp0.wait()
        cp1.wait()

        def row_body(i, carry2):
            def col_body(j, carry3):
                r0_v[i, pl.ds(j * 16, 16)] = (
                    r0_v[i, pl.ds(j * 16, 16)] + r1_v[i, pl.ds(j * 16, 16)])
                return carry3
            lax.fori_loop(0, HIDDEN // 16, col_body, 0)
            return carry2

        lax.fori_loop(0, ECH, row_body, 0)
        pltpu.sync_copy(r0_v, out_hbm.at[pl.ds(tb + ci * ECH, ECH)])
        return carry

    lax.fori_loop(0, TCH // ECH, chunk_body, 0)


_combine_impl = None


def _combine(pos0f, pos1f, y):
    global _combine_impl
    if _combine_impl is None:
        _combine_impl = pl.kernel(
            _combine_body,
            out_type=jax.ShapeDtypeStruct((T, HIDDEN), jnp.float32),
            mesh=_mesh(),
            scratch_types=[
                pltpu.VMEM((TCH,), jnp.int32),
                pltpu.VMEM((TCH,), jnp.int32),
                pltpu.VMEM((ECH, HIDDEN), jnp.float32),
                pltpu.VMEM((ECH, HIDDEN), jnp.float32),
                pltpu.SemaphoreType.DMA,
                pltpu.SemaphoreType.DMA,
            ],
            compiler_params=pltpu.CompilerParams(needs_layout_passes=False),
        )
    return _combine_impl(pos0f, pos1f, y)


def kernel(hidden_states, router_logits, w13_weight, w2_weight):
    x16 = hidden_states.astype(jnp.bfloat16)

    pos0, pos1, wp0, wp1, be, ba, xb = _route(router_logits)
    y = _gmm(be.reshape(NB), ba.reshape(NB), xb.reshape(NB),
             pos0, pos1, wp0, wp1, x16, w13_weight, w2_weight)
    return _combine(pos0.reshape(T), pos1.reshape(T), y)
